# baseline (speedup 1.0000x reference)
import jax
import jax.numpy as jnp
from jax import lax
from jax.experimental import pallas as pl
from jax.experimental.pallas import tpu as pltpu
from jax.experimental.pallas import tpu_sc as plsc

N_NODES = 10000
N_EDGES = 320000
D = 128

NC = 2
NS = 16
OWN = 640
ACC_ROWS = OWN + 8
SCAN = 2048
SCAN_CHUNKS = 80
E_PAD = NC * SCAN_CHUNKS * SCAN
PAD_DST = 16384
GRP = 128
PEND_CAP = 2304
VEC = 16
CAP = 164096


def _sc_body(x_hbm, src_hbm, dst_hbm, out_hbm, srcl_hbm, dstl_hbm,
             acc, sin, pend_p, gbuf_s, gbuf_d, ibuf, stag,
             csem, wsem0, wsem1, ilsem0, ilsem1, gsem0, gsem1):
    wsem = [wsem0, wsem1]
    ilsem = [ilsem0, ilsem1]
    gsem = [gsem0, gsem1]
    c = lax.axis_index("c")
    s = lax.axis_index("s")
    lo = s * OWN
    wbase = (c * NS + s) * CAP

    zv = jnp.zeros((VEC,), jnp.float32)
    def _zrow(r, _):
        for g in range(D // VEC):
            acc[pl.ds(D * r + VEC * g, VEC)] = zv
        return 0
    lax.fori_loop(0, ACC_ROWS, _zrow, 0)

    def _dec(base, b2):
        def _d(m, _):
            pv = pend_p[pl.ds(base + VEC * m, VEC)]
            gbuf_s[b2, pl.ds(VEC * m, VEC)] = lax.shift_right_logical(pv, 10)
            gbuf_d[b2, pl.ds(VEC * m, VEC)] = pv & jnp.int32(1023)
            return 0
        lax.fori_loop(0, GRP // VEC, _d, 0)

    pltpu.sync_copy(src_hbm.at[c, 0], sin.at[0, 0])
    pltpu.sync_copy(dst_hbm.at[c, 0], sin.at[0, 1])

    def _chunk(i, carry):
        off, gtot = carry
        for b in range(2):
            cs = 2 * i + b
            @pl.when(cs + 1 < SCAN_CHUNKS)
            def _():
                pltpu.async_copy(src_hbm.at[c, cs + 1], sin.at[1 - b, 0], csem)
                pltpu.async_copy(dst_hbm.at[c, cs + 1], sin.at[1 - b, 1], csem)
            def _scan8(t, off):
                for u in range(8):
                    v = 8 * t + u
                    dv = sin[b, 1, pl.ds(VEC * v, VEC)]
                    sv = sin[b, 0, pl.ds(VEC * v, VEC)]
                    dlv = dv - lo
                    msk = (dlv >= 0) & (dlv < OWN)
                    key = jnp.where(msk, jnp.int32(0), jnp.int32(1))
                    packed = lax.shift_left(sv, 10) | (dlv & jnp.int32(1023))
                    _, pvec = plsc.sort_key_val(key, packed)
                    pend_p[pl.ds(off, VEC)] = pvec
                    cntv = plsc.all_reduce_population_count(msk)
                    off = off + cntv[0]
                return off
            off = lax.fori_loop(0, SCAN // VEC // 8, _scan8, off)
            n_full = off // GRP
            def _wpair(i2, _):
                for b2 in range(2):
                    t = 2 * i2 + b2
                    @pl.when(t < n_full)
                    def _():
                        @pl.when(t >= 2)
                        def _():
                            pltpu.make_async_copy(
                                gbuf_s.at[b2], srcl_hbm.at[pl.ds(0, GRP)],
                                wsem[b2]).wait()
                            pltpu.make_async_copy(
                                gbuf_d.at[b2], dstl_hbm.at[pl.ds(0, GRP)],
                                wsem[b2]).wait()
                        _dec(GRP * t, b2)
                        wo = wbase + (gtot + t) * GRP
                        pltpu.async_copy(gbuf_s.at[b2],
                                         srcl_hbm.at[pl.ds(wo, GRP)], wsem[b2])
                        pltpu.async_copy(gbuf_d.at[b2],
                                         dstl_hbm.at[pl.ds(wo, GRP)], wsem[b2])
                return 0
            lax.fori_loop(0, (n_full + 1) // 2, _wpair, 0)
            @pl.when(n_full >= 1)
            def _():
                pltpu.make_async_copy(gbuf_s.at[0], srcl_hbm.at[pl.ds(0, GRP)],
                                      wsem[0]).wait()
                pltpu.make_async_copy(gbuf_d.at[0], dstl_hbm.at[pl.ds(0, GRP)],
                                      wsem[0]).wait()
            @pl.when(n_full >= 2)
            def _():
                pltpu.make_async_copy(gbuf_s.at[1], srcl_hbm.at[pl.ds(0, GRP)],
                                      wsem[1]).wait()
                pltpu.make_async_copy(gbuf_d.at[1], dstl_hbm.at[pl.ds(0, GRP)],
                                      wsem[1]).wait()
            rem_base = GRP * n_full
            @pl.when(n_full > 0)
            def _():
                for u in range(GRP // VEC + 1):
                    pend_p[pl.ds(VEC * u, VEC)] = pend_p[pl.ds(rem_base + VEC * u, VEC)]
            gtot = gtot + n_full
            off = off - GRP * n_full
            @pl.when(cs + 1 < SCAN_CHUNKS)
            def _():
                pltpu.make_async_copy(src_hbm.at[c, 0], sin.at[1 - b, 0],
                                      csem).wait()
                pltpu.make_async_copy(dst_hbm.at[c, 0], sin.at[1 - b, 1],
                                      csem).wait()
        return (off, gtot)

    off, gtot = lax.fori_loop(0, SCAN_CHUNKS // 2, _chunk,
                              (jnp.int32(0), jnp.int32(0)))

    @pl.when(off > 0)
    def _():
        di = jnp.full((VEC,), OWN, jnp.int32)
        for u in range(GRP // VEC + 1):
            pend_p[pl.ds(off + VEC * u, VEC)] = di
        _dec(0, 0)
        wo = wbase + gtot * GRP
        pltpu.sync_copy(gbuf_s.at[0], srcl_hbm.at[pl.ds(wo, GRP)])
        pltpu.sync_copy(gbuf_d.at[0], dstl_hbm.at[pl.ds(wo, GRP)])
    ng = jnp.where(off > 0, gtot + 1, gtot)

    def _loads(t, b2):
        wo = wbase + t * GRP
        pltpu.async_copy(srcl_hbm.at[pl.ds(wo, GRP)],
                         ibuf.at[pl.ds(2 * GRP * b2, GRP)], ilsem[b2])
        pltpu.async_copy(dstl_hbm.at[pl.ds(wo, GRP)],
                         ibuf.at[pl.ds(2 * GRP * b2 + GRP, GRP)], ilsem[b2])

    def _wait_loads(b2):
        pltpu.make_async_copy(srcl_hbm.at[pl.ds(0, GRP)],
                              ibuf.at[pl.ds(2 * GRP * b2, GRP)], ilsem[b2]).wait()
        pltpu.make_async_copy(dstl_hbm.at[pl.ds(0, GRP)],
                              ibuf.at[pl.ds(2 * GRP * b2 + GRP, GRP)], ilsem[b2]).wait()

    def _add(b2):
        iot = lax.iota(jnp.int32, VEC)
        def _row4(q, _):
            for e in range(4):
                r = 4 * q + e
                posv = jnp.full((VEC,), 2 * GRP * b2 + GRP, jnp.int32) + r
                dlb = plsc.load_gather(ibuf, [posv])
                fb = lax.shift_left(dlb, 7)
                for g in range(D // VEC):
                    idxg = fb + (iot + VEC * g)
                    vals = stag[b2, r, pl.ds(VEC * g, VEC)]
                    plsc.addupdate_scatter(acc, [idxg], vals)
            return 0
        lax.fori_loop(0, GRP // 4, _row4, 0)

    @pl.when(ng > 0)
    def _():
        _loads(0, 0)
    @pl.when(ng > 1)
    def _():
        _loads(1, 1)
    @pl.when(ng > 0)
    def _():
        _wait_loads(0)
        pltpu.async_copy(x_hbm.at[ibuf.at[pl.ds(0, GRP)]], stag.at[0], gsem[0])

    def _bpair(i2, _):
        for b2 in range(2):
            t = 2 * i2 + b2
            @pl.when(t < ng)
            def _():
                @pl.when(t + 1 < ng)
                def _():
                    _wait_loads(1 - b2)
                    pltpu.async_copy(x_hbm.at[ibuf.at[pl.ds(2 * GRP * (1 - b2), GRP)]],
                                     stag.at[1 - b2], gsem[1 - b2])
                pltpu.make_async_copy(x_hbm.at[pl.ds(0, GRP)], stag.at[b2],
                                      gsem[b2]).wait()
                _add(b2)
                @pl.when(t + 2 < ng)
                def _():
                    _loads(t + 2, b2)
        return 0

    lax.fori_loop(0, (ng + 1) // 2, _bpair, 0)

    tail = N_NODES - (NS - 1) * OWN
    obase = c * (N_NODES * D)
    @pl.when(s < NS - 1)
    def _():
        pltpu.sync_copy(acc.at[pl.ds(0, OWN * D)],
                        out_hbm.at[pl.ds(obase + lo * D, OWN * D)])
    @pl.when(s == NS - 1)
    def _():
        pltpu.sync_copy(acc.at[pl.ds(0, tail * D)],
                        out_hbm.at[pl.ds(obase + (NS - 1) * OWN * D, tail * D)])


def _tc_add_body(p_ref, o_ref):
    o_ref[...] = p_ref[0] + p_ref[1]


@jax.jit
def _run(x, edge_index):
    n_pad = E_PAD - N_EDGES
    src_p = jnp.concatenate([edge_index[0], jnp.zeros((n_pad,), jnp.int32)])
    dst_p = jnp.concatenate([edge_index[1], jnp.full((n_pad,), PAD_DST, jnp.int32)])
    src3 = src_p.reshape(NC, SCAN_CHUNKS, SCAN)
    dst3 = dst_p.reshape(NC, SCAN_CHUNKS, SCAN)

    mesh = plsc.VectorSubcoreMesh(core_axis_name="c", subcore_axis_name="s")
    partials, _, _ = pl.kernel(
        _sc_body,
        out_type=(jax.ShapeDtypeStruct((NC * N_NODES * D,), jnp.float32),
                  jax.ShapeDtypeStruct((NC * NS * CAP,), jnp.int32),
                  jax.ShapeDtypeStruct((NC * NS * CAP,), jnp.int32)),
        mesh=mesh,
        compiler_params=pltpu.CompilerParams(needs_layout_passes=False),
        scratch_types=[
            pltpu.VMEM((ACC_ROWS * D,), jnp.float32),
            pltpu.VMEM((2, 2, SCAN), jnp.int32),
            pltpu.VMEM((PEND_CAP,), jnp.int32),
            pltpu.VMEM((2, GRP), jnp.int32),
            pltpu.VMEM((2, GRP), jnp.int32),
            pltpu.VMEM((2 * 2 * GRP,), jnp.int32),
            pltpu.VMEM((2, GRP, D), jnp.float32),
            pltpu.SemaphoreType.DMA,
            pltpu.SemaphoreType.DMA,
            pltpu.SemaphoreType.DMA,
            pltpu.SemaphoreType.DMA,
            pltpu.SemaphoreType.DMA,
            pltpu.SemaphoreType.DMA,
            pltpu.SemaphoreType.DMA,
        ],
    )(x, src3, dst3)

    p2 = partials.reshape(NC, N_NODES * D)
    out = pl.pallas_call(
        _tc_add_body,
        out_shape=jax.ShapeDtypeStruct((N_NODES * D,), jnp.float32),
        grid=(10,),
        in_specs=[pl.BlockSpec((NC, N_NODES * D // 10), lambda i: (0, i))],
        out_specs=pl.BlockSpec((N_NODES * D // 10,), lambda i: (i,)),
    )(p2)
    return out.reshape(N_NODES, D)


def kernel(t, x, edge_index):
    return _run(x, edge_index)


# speedup vs baseline: 1.0932x; 1.0932x over previous
import jax
import jax.numpy as jnp
from jax import lax
from jax.experimental import pallas as pl
from jax.experimental.pallas import tpu as pltpu
from jax.experimental.pallas import tpu_sc as plsc

N_NODES = 10000
N_EDGES = 320000
D = 128

NC = 2
NS = 16
OWN = 640
ACC_ROWS = OWN + 8
SCAN = 2048
SCAN_CHUNKS = 80
E_PAD = NC * SCAN_CHUNKS * SCAN
PAD_DST = 16384
GRP = 128
PEND_CAP = 2304
VEC = 16
CAP = 164096


def _sc_body(x_hbm, src_hbm, dst_hbm, out_hbm, srcl_hbm, dstl_hbm,
             acc, sin, pend_p, gbuf_s, gbuf_d, ibuf, stag,
             csem, wsem0, wsem1, ilsem0, ilsem1, gsem0, gsem1):
    wsem = [wsem0, wsem1]
    ilsem = [ilsem0, ilsem1]
    gsem = [gsem0, gsem1]
    c = lax.axis_index("c")
    s = lax.axis_index("s")
    lo = s * OWN
    wbase = (c * NS + s) * CAP

    zv = jnp.zeros((VEC,), jnp.float32)
    def _zrow(r, _):
        for g in range(D // VEC):
            acc[r, pl.ds(VEC * g, VEC)] = zv
        return 0
    lax.fori_loop(0, ACC_ROWS, _zrow, 0)

    def _dec(base, b2):
        def _d(m, _):
            pv = pend_p[pl.ds(base + VEC * m, VEC)]
            gbuf_s[b2, pl.ds(VEC * m, VEC)] = lax.shift_right_logical(pv, 10)
            gbuf_d[b2, pl.ds(VEC * m, VEC)] = pv & jnp.int32(1023)
            return 0
        lax.fori_loop(0, GRP // VEC, _d, 0)

    pltpu.sync_copy(src_hbm.at[c, 0], sin.at[0, 0])
    pltpu.sync_copy(dst_hbm.at[c, 0], sin.at[0, 1])

    def _chunk(i, carry):
        off, gtot = carry
        for b in range(2):
            cs = 2 * i + b
            @pl.when(cs + 1 < SCAN_CHUNKS)
            def _():
                pltpu.async_copy(src_hbm.at[c, cs + 1], sin.at[1 - b, 0], csem)
                pltpu.async_copy(dst_hbm.at[c, cs + 1], sin.at[1 - b, 1], csem)
            def _scan8(t, off):
                for u in range(8):
                    v = 8 * t + u
                    dv = sin[b, 1, pl.ds(VEC * v, VEC)]
                    sv = sin[b, 0, pl.ds(VEC * v, VEC)]
                    dlv = dv - lo
                    msk = (dlv >= 0) & (dlv < OWN)
                    key = jnp.where(msk, jnp.int32(0), jnp.int32(1))
                    packed = lax.shift_left(sv, 10) | (dlv & jnp.int32(1023))
                    _, pvec = plsc.sort_key_val(key, packed)
                    pend_p[pl.ds(off, VEC)] = pvec
                    cntv = plsc.all_reduce_population_count(msk)
                    off = off + cntv[0]
                return off
            off = lax.fori_loop(0, SCAN // VEC // 8, _scan8, off)
            n_full = off // GRP
            def _wpair(i2, _):
                for b2 in range(2):
                    t = 2 * i2 + b2
                    @pl.when(t < n_full)
                    def _():
                        @pl.when(t >= 2)
                        def _():
                            pltpu.make_async_copy(
                                gbuf_s.at[b2], srcl_hbm.at[pl.ds(0, GRP)],
                                wsem[b2]).wait()
                            pltpu.make_async_copy(
                                gbuf_d.at[b2], dstl_hbm.at[pl.ds(0, GRP)],
                                wsem[b2]).wait()
                        _dec(GRP * t, b2)
                        wo = wbase + (gtot + t) * GRP
                        pltpu.async_copy(gbuf_s.at[b2],
                                         srcl_hbm.at[pl.ds(wo, GRP)], wsem[b2])
                        pltpu.async_copy(gbuf_d.at[b2],
                                         dstl_hbm.at[pl.ds(wo, GRP)], wsem[b2])
                return 0
            lax.fori_loop(0, (n_full + 1) // 2, _wpair, 0)
            @pl.when(n_full >= 1)
            def _():
                pltpu.make_async_copy(gbuf_s.at[0], srcl_hbm.at[pl.ds(0, GRP)],
                                      wsem[0]).wait()
                pltpu.make_async_copy(gbuf_d.at[0], dstl_hbm.at[pl.ds(0, GRP)],
                                      wsem[0]).wait()
            @pl.when(n_full >= 2)
            def _():
                pltpu.make_async_copy(gbuf_s.at[1], srcl_hbm.at[pl.ds(0, GRP)],
                                      wsem[1]).wait()
                pltpu.make_async_copy(gbuf_d.at[1], dstl_hbm.at[pl.ds(0, GRP)],
                                      wsem[1]).wait()
            rem_base = GRP * n_full
            @pl.when(n_full > 0)
            def _():
                for u in range(GRP // VEC + 1):
                    pend_p[pl.ds(VEC * u, VEC)] = pend_p[pl.ds(rem_base + VEC * u, VEC)]
            gtot = gtot + n_full
            off = off - GRP * n_full
            @pl.when(cs + 1 < SCAN_CHUNKS)
            def _():
                pltpu.make_async_copy(src_hbm.at[c, 0], sin.at[1 - b, 0],
                                      csem).wait()
                pltpu.make_async_copy(dst_hbm.at[c, 0], sin.at[1 - b, 1],
                                      csem).wait()
        return (off, gtot)

    off, gtot = lax.fori_loop(0, SCAN_CHUNKS // 2, _chunk,
                              (jnp.int32(0), jnp.int32(0)))

    @pl.when(off > 0)
    def _():
        di = jnp.full((VEC,), OWN, jnp.int32)
        for u in range(GRP // VEC + 1):
            pend_p[pl.ds(off + VEC * u, VEC)] = di
        _dec(0, 0)
        wo = wbase + gtot * GRP
        pltpu.sync_copy(gbuf_s.at[0], srcl_hbm.at[pl.ds(wo, GRP)])
        pltpu.sync_copy(gbuf_d.at[0], dstl_hbm.at[pl.ds(wo, GRP)])
    ng = jnp.where(off > 0, gtot + 1, gtot)

    def _loads(t, b2):
        wo = wbase + t * GRP
        pltpu.async_copy(srcl_hbm.at[pl.ds(wo, GRP)], ibuf.at[b2, 0], ilsem[b2])
        pltpu.async_copy(dstl_hbm.at[pl.ds(wo, GRP)], ibuf.at[b2, 1], ilsem[b2])

    def _wait_loads(b2):
        pltpu.make_async_copy(srcl_hbm.at[pl.ds(0, GRP)], ibuf.at[b2, 0],
                              ilsem[b2]).wait()
        pltpu.make_async_copy(dstl_hbm.at[pl.ds(0, GRP)], ibuf.at[b2, 1],
                              ilsem[b2]).wait()

    def _add(b2):
        def _sub(m, _):
            dlv = ibuf[b2, 1, pl.ds(VEC * m, VEC)]
            for k in range(VEC):
                dl = dlv[k]
                for g in range(D // VEC):
                    plsc.addupdate(acc.at[dl, pl.ds(VEC * g, VEC)],
                                   stag[b2, VEC * m + k, pl.ds(VEC * g, VEC)])
            return 0
        lax.fori_loop(0, GRP // VEC, _sub, 0)

    @pl.when(ng > 0)
    def _():
        _loads(0, 0)
    @pl.when(ng > 1)
    def _():
        _loads(1, 1)
    def _gather4(b2):
        for h in range(4):
            pltpu.async_copy(x_hbm.at[ibuf.at[b2, 0, pl.ds(32 * h, 32)]],
                             stag.at[b2, pl.ds(32 * h, 32)], gsem[b2])

    def _wait_gather4(b2):
        for h in range(4):
            pltpu.make_async_copy(x_hbm.at[pl.ds(0, 32)],
                                  stag.at[b2, pl.ds(32 * h, 32)],
                                  gsem[b2]).wait()

    @pl.when(ng > 0)
    def _():
        _wait_loads(0)
        _gather4(0)

    def _bpair(i2, _):
        for b2 in range(2):
            t = 2 * i2 + b2
            @pl.when(t < ng)
            def _():
                @pl.when(t + 1 < ng)
                def _():
                    _wait_loads(1 - b2)
                    _gather4(1 - b2)
                _wait_gather4(b2)
                _add(b2)
                @pl.when(t + 2 < ng)
                def _():
                    _loads(t + 2, b2)
        return 0

    lax.fori_loop(0, (ng + 1) // 2, _bpair, 0)

    @pl.when(s < NS - 1)
    def _():
        pltpu.sync_copy(acc.at[pl.ds(0, OWN)], out_hbm.at[c, pl.ds(lo, OWN)])
    @pl.when(s == NS - 1)
    def _():
        pltpu.sync_copy(acc.at[pl.ds(0, N_NODES - (NS - 1) * OWN)],
                        out_hbm.at[c, pl.ds((NS - 1) * OWN,
                                            N_NODES - (NS - 1) * OWN)])


def _tc_add_body(p_ref, o_ref):
    o_ref[...] = p_ref[0] + p_ref[1]


@jax.jit
def _run(x, edge_index):
    n_pad = E_PAD - N_EDGES
    src_p = jnp.concatenate([edge_index[0], jnp.zeros((n_pad,), jnp.int32)])
    dst_p = jnp.concatenate([edge_index[1], jnp.full((n_pad,), PAD_DST, jnp.int32)])
    src3 = src_p.reshape(NC, SCAN_CHUNKS, SCAN)
    dst3 = dst_p.reshape(NC, SCAN_CHUNKS, SCAN)

    mesh = plsc.VectorSubcoreMesh(core_axis_name="c", subcore_axis_name="s")
    partials, _, _ = pl.kernel(
        _sc_body,
        out_type=(jax.ShapeDtypeStruct((NC, N_NODES, D), jnp.float32),
                  jax.ShapeDtypeStruct((NC * NS * CAP,), jnp.int32),
                  jax.ShapeDtypeStruct((NC * NS * CAP,), jnp.int32)),
        mesh=mesh,
        compiler_params=pltpu.CompilerParams(needs_layout_passes=False),
        scratch_types=[
            pltpu.VMEM((ACC_ROWS, D), jnp.float32),
            pltpu.VMEM((2, 2, SCAN), jnp.int32),
            pltpu.VMEM((PEND_CAP,), jnp.int32),
            pltpu.VMEM((2, GRP), jnp.int32),
            pltpu.VMEM((2, GRP), jnp.int32),
            pltpu.VMEM((2, 2, GRP), jnp.int32),
            pltpu.VMEM((2, GRP, D), jnp.float32),
            pltpu.SemaphoreType.DMA,
            pltpu.SemaphoreType.DMA,
            pltpu.SemaphoreType.DMA,
            pltpu.SemaphoreType.DMA,
            pltpu.SemaphoreType.DMA,
            pltpu.SemaphoreType.DMA,
            pltpu.SemaphoreType.DMA,
        ],
    )(x, src3, dst3)

    out = pl.pallas_call(
        _tc_add_body,
        out_shape=jax.ShapeDtypeStruct((N_NODES, D), jnp.float32),
        grid=(10,),
        in_specs=[pl.BlockSpec((NC, N_NODES // 10, D), lambda i: (0, i, 0))],
        out_specs=pl.BlockSpec((N_NODES // 10, D), lambda i: (i, 0)),
    )(partials)
    return out


def kernel(t, x, edge_index):
    return _run(x, edge_index)


# speedup vs baseline: 1.2340x; 1.1288x over previous
import jax
import jax.numpy as jnp
from jax import lax
from jax.experimental import pallas as pl
from jax.experimental.pallas import tpu as pltpu
from jax.experimental.pallas import tpu_sc as plsc

N_NODES = 10000
N_EDGES = 320000
D = 128

NC = 2
NS = 16
NW = NC * NS
CHUNK = 128
CHUNKS_PER_W = 80
E_PAD = NW * CHUNKS_PER_W * CHUNK
ACC_ROWS = 10240
PAD_ROW = N_NODES + 8
ZCOPIES = 5
IDX_DEPTH = 4
ROWS_PER_TILE_OUT = 624
OUT_TAIL = N_NODES - NS * ROWS_PER_TILE_OUT


def _sc_body(x_hbm, eidx_hbm, out_hbm,
             acc_sh, idxs, rows,
             gsem0, gsem1, isem0, isem1, isem2, isem3):
    gsem = [gsem0, gsem1]
    isem = [isem0, isem1, isem2, isem3]
    c = lax.axis_index("c")
    s = lax.axis_index("s")
    base = (c * NS + s) * CHUNKS_PER_W

    def _zrow(r, _):
        for k in range(D // 16):
            rows[0, r, pl.ds(16 * k, 16)] = jnp.zeros((16,), jnp.float32)
        return 0
    lax.fori_loop(0, CHUNK, _zrow, 0)
    for q in range(ZCOPIES):
        pltpu.sync_copy(rows.at[0],
                        acc_sh.at[pl.ds(s * (ZCOPIES * CHUNK) + q * CHUNK, CHUNK)])

    pltpu.sync_copy(eidx_hbm.at[base], idxs.at[0])
    pltpu.async_copy(x_hbm.at[idxs.at[0, 0]], rows.at[0], gsem[0])
    for p in range(1, IDX_DEPTH):
        pltpu.async_copy(eidx_hbm.at[base + p], idxs.at[p], isem[p])

    plsc.subcore_barrier()

    def _quad(i, _):
        for b in range(IDX_DEPTH):
            jb = IDX_DEPTH * i + b
            pn = (b + 1) % IDX_DEPTH
            rn = (b + 1) % 2
            @pl.when(jb + 1 < CHUNKS_PER_W)
            def _():
                pltpu.make_async_copy(eidx_hbm.at[base], idxs.at[pn],
                                      isem[pn]).wait()
                pltpu.async_copy(x_hbm.at[idxs.at[pn, 0]], rows.at[rn],
                                 gsem[rn])
            pltpu.make_async_copy(x_hbm.at[pl.ds(0, CHUNK)], rows.at[b % 2],
                                  gsem[b % 2]).wait()
            pltpu.sync_copy(rows.at[b % 2], acc_sh.at[idxs.at[b, 1]], add=True)
            @pl.when(jb + IDX_DEPTH < CHUNKS_PER_W)
            def _():
                pltpu.async_copy(eidx_hbm.at[base + jb + IDX_DEPTH],
                                 idxs.at[b], isem[b])
        return 0

    lax.fori_loop(0, CHUNKS_PER_W // IDX_DEPTH, _quad, 0)

    plsc.subcore_barrier()

    pltpu.sync_copy(acc_sh.at[pl.ds(s * ROWS_PER_TILE_OUT, ROWS_PER_TILE_OUT)],
                    out_hbm.at[c, pl.ds(s * ROWS_PER_TILE_OUT, ROWS_PER_TILE_OUT)])

    @pl.when(s == 0)
    def _():
        pltpu.sync_copy(acc_sh.at[pl.ds(NS * ROWS_PER_TILE_OUT, OUT_TAIL)],
                        out_hbm.at[c, pl.ds(NS * ROWS_PER_TILE_OUT, OUT_TAIL)])


def _tc_add_body(p_ref, o_ref):
    o_ref[...] = p_ref[0] + p_ref[1]


@jax.jit
def _run(x, edge_index):
    n_pad = E_PAD - N_EDGES
    src_p = jnp.concatenate([edge_index[0], jnp.zeros((n_pad,), jnp.int32)])
    dst_p = jnp.concatenate([edge_index[1], jnp.full((n_pad,), PAD_ROW, jnp.int32)])
    eidx = jnp.stack([src_p.reshape(NW * CHUNKS_PER_W, CHUNK),
                      dst_p.reshape(NW * CHUNKS_PER_W, CHUNK)], axis=1)

    mesh = plsc.VectorSubcoreMesh(core_axis_name="c", subcore_axis_name="s")
    partials = pl.kernel(
        _sc_body,
        out_type=jax.ShapeDtypeStruct((NC, N_NODES, D), jnp.float32),
        mesh=mesh,
        scratch_types=[
            pltpu.VMEM_SHARED((ACC_ROWS, D), jnp.float32),
            pltpu.VMEM((IDX_DEPTH, 2, CHUNK), jnp.int32),
            pltpu.VMEM((2, CHUNK, D), jnp.float32),
            pltpu.SemaphoreType.DMA,
            pltpu.SemaphoreType.DMA,
            pltpu.SemaphoreType.DMA,
            pltpu.SemaphoreType.DMA,
            pltpu.SemaphoreType.DMA,
            pltpu.SemaphoreType.DMA,
        ],
    )(x, eidx)

    out = pl.pallas_call(
        _tc_add_body,
        out_shape=jax.ShapeDtypeStruct((N_NODES, D), jnp.float32),
        grid=(10,),
        in_specs=[pl.BlockSpec((NC, N_NODES // 10, D), lambda i: (0, i, 0))],
        out_specs=pl.BlockSpec((N_NODES // 10, D), lambda i: (i, 0)),
    )(partials)
    return out


def kernel(t, x, edge_index):
    return _run(x, edge_index)
